# Initial kernel scaffold; baseline (speedup 1.0000x reference)
import jax
import jax.numpy as jnp
from jax.experimental import pallas as pl


def kernel(x, edge_attr, edge_index, batch, Wi, bi, att1_W, att1_b, att2_W, att2_b, tr_W, tr_b, gru_Wih, gru_Whh, gru_bih, gru_bhh, ln_w, ln_b, lin_W0, lin_b0, ee1_W0, ee1_b0, ee2_W0, ee2_b0, tg1_W0, tg1_b0, tg2_W0, tg2_b0, lin_W1, lin_b1, ee1_W1, ee1_b1, ee2_W1, ee2_b1, tg1_W1, tg1_b1, tg2_W1, tg2_b1):
    raise NotImplementedError("write your pallas kernel here")


# speedup vs baseline: 2.6442x; 2.6442x over previous
import functools

import jax
import jax.numpy as jnp
from jax import lax
from jax.experimental import pallas as pl
from jax.experimental.pallas import tpu as pltpu
from jax.experimental.pallas import tpu_sc as plsc

N = 10000
E = 160000
DN = 256
DE = 16
H = 32
O = 256
B = 8

NP = 10240
EP = 163840
NW = 32
EW = EP // NW
WIN = 640
NWIN = EW // WIN
SHARE = NP // 16
NP4 = NP // 4
SHARE4 = NP4 // 16

def _mesh():
    return plsc.VectorSubcoreMesh(core_axis_name="c", subcore_axis_name="s")

f32 = jnp.float32


def _zero_rows(ref, rows, cols):
    z16 = jnp.zeros((16,), f32)

    def body(i, _):
        for c in range(cols // 16):
            ref[i, pl.ds(c * 16, 16)] = z16
        return 0

    lax.fori_loop(0, rows, body, 0)


def _fill_ones(ref, n):
    o16 = jnp.ones((16,), f32)

    def body(i, _):
        ref[pl.ds(i * 16, 16)] = o16
        return 0

    lax.fori_loop(0, n // 16, body, 0)


def _zero_flat(ref, n):
    z16 = jnp.zeros((16,), f32)

    def body(i, _):
        ref[pl.ds(i * 16, 16)] = z16
        return 0

    lax.fori_loop(0, n // 16, body, 0)


@functools.lru_cache(None)
def _sc_degrees_k():
  return functools.partial(
    pl.kernel,
    out_type=jax.ShapeDtypeStruct((2, 2, NP), f32),
    mesh=_mesh(),
    scratch_types=[
        pltpu.VMEM((WIN,), jnp.int32),
        pltpu.VMEM((WIN,), f32),
        pltpu.VMEM((SHARE,), f32),
        pltpu.VMEM_SHARED((NP,), f32),
        pltpu.VMEM_SHARED((NP,), f32),
    ],
  )(_sc_degrees_body)


def _sc_degrees(dstp, srcp):
    return _sc_degrees_k()(dstp, srcp)


def _sc_degrees_body(dst_hbm, src_hbm, out_hbm, idx_w, ones_w, zb, sh_i, sh_j):
    cid = lax.axis_index("c")
    sid = lax.axis_index("s")
    wid = sid * 2 + cid
    base = wid * EW
    _fill_ones(ones_w, WIN)
    _zero_flat(zb, SHARE)
    pltpu.sync_copy(zb, sh_i.at[pl.ds(sid * SHARE, SHARE)])
    pltpu.sync_copy(zb, sh_j.at[pl.ds(sid * SHARE, SHARE)])
    plsc.subcore_barrier()
    for w in range(NWIN):
        pltpu.sync_copy(dst_hbm.at[pl.ds(base + w * WIN, WIN)], idx_w)
        pltpu.sync_copy(ones_w, sh_i.at[idx_w], add=True)
        pltpu.sync_copy(src_hbm.at[pl.ds(base + w * WIN, WIN)], idx_w)
        pltpu.sync_copy(ones_w, sh_j.at[idx_w], add=True)
    plsc.subcore_barrier()
    pltpu.sync_copy(sh_i.at[pl.ds(sid * SHARE, SHARE)],
                    out_hbm.at[cid, 0, pl.ds(sid * SHARE, SHARE)])
    pltpu.sync_copy(sh_j.at[pl.ds(sid * SHARE, SHARE)],
                    out_hbm.at[cid, 1, pl.ds(sid * SHARE, SHARE)])


@functools.lru_cache(None)
def _sc_gather0_k():
  return functools.partial(
    pl.kernel,
    out_type=(
        jax.ShapeDtypeStruct((EP, 128), f32),
        jax.ShapeDtypeStruct((EP,), f32),
        jax.ShapeDtypeStruct((EP,), f32),
    ),
    mesh=_mesh(),
    scratch_types=[
        pltpu.VMEM((WIN,), jnp.int32),
        pltpu.VMEM((WIN,), jnp.int32),
        pltpu.VMEM((WIN, 128), f32),
        pltpu.VMEM((WIN,), f32),
        pltpu.VMEM((WIN,), f32),
        pltpu.VMEM((WIN,), f32),
        pltpu.VMEM((WIN,), f32),
        pltpu.SemaphoreType.DMA,
    ],
  )(_sc_gather0_body)


def _sc_gather0(hlp, srcp, dstp, degi, degj):
    return _sc_gather0_k()(hlp, srcp, dstp, degi, degj)


def _sc_gather0_body(hl_hbm, src_hbm, dst_hbm, degi_hbm, degj_hbm,
                xj_hbm, inv0_hbm, inv1_hbm,
                idxs, idxd, rows, a_w, b_w, inv0_w, inv1_w, sem):
    cid = lax.axis_index("c")
    sid = lax.axis_index("s")
    wid = sid * 2 + cid
    base = wid * EW
    for w in range(NWIN):
        lo = base + w * WIN
        pltpu.sync_copy(src_hbm.at[pl.ds(lo, WIN)], idxs)
        pltpu.sync_copy(dst_hbm.at[pl.ds(lo, WIN)], idxd)
        pltpu.async_copy(hl_hbm.at[idxs], rows, sem).wait()
        pltpu.async_copy(degi_hbm.at[idxd], a_w, sem).wait()
        pltpu.async_copy(degj_hbm.at[idxs], b_w, sem).wait()

        def body(i, _):
            a = a_w[pl.ds(i * 16, 16)]
            b = b_w[pl.ds(i * 16, 16)]
            s = a + b + 1e-8
            inv0_w[pl.ds(i * 16, 16)] = a / s
            inv1_w[pl.ds(i * 16, 16)] = b / s
            return 0

        lax.fori_loop(0, WIN // 16, body, 0)
        pltpu.sync_copy(rows, xj_hbm.at[pl.ds(lo, WIN)])
        pltpu.sync_copy(inv0_w, inv0_hbm.at[pl.ds(lo, WIN)])
        pltpu.sync_copy(inv1_w, inv1_hbm.at[pl.ds(lo, WIN)])


@functools.lru_cache(None)
def _sc_gather1_k():
  return functools.partial(
    pl.kernel,
    out_type=jax.ShapeDtypeStruct((EP, 128), f32),
    mesh=_mesh(),
    scratch_types=[
        pltpu.VMEM((WIN,), jnp.int32),
        pltpu.VMEM((WIN, 128), f32),
        pltpu.SemaphoreType.DMA,
    ],
  )(_sc_gather1_body)


def _sc_gather1(hlp, srcp):
    return _sc_gather1_k()(hlp, srcp)


def _sc_gather1_body(hl_hbm, src_hbm, xj_hbm, idxs, rows, sem):
    cid = lax.axis_index("c")
    sid = lax.axis_index("s")
    wid = sid * 2 + cid
    base = wid * EW
    for w in range(NWIN):
        lo = base + w * WIN
        pltpu.sync_copy(src_hbm.at[pl.ds(lo, WIN)], idxs)
        pltpu.async_copy(hl_hbm.at[idxs], rows, sem).wait()
        pltpu.sync_copy(rows, xj_hbm.at[pl.ds(lo, WIN)])


@functools.lru_cache(None)
def _sc_scatter_k():
  return functools.partial(
    pl.kernel,
    out_type=jax.ShapeDtypeStruct((2, NP4, 128), f32),
    mesh=_mesh(),
    scratch_types=[
        pltpu.VMEM((WIN,), jnp.int32),
        pltpu.VMEM((WIN, 128), f32),
        pltpu.VMEM((SHARE4, 128), f32),
        pltpu.VMEM_SHARED((NP4, 128), f32),
    ],
  )(_sc_scatter_body)


def _sc_scatter(msgp, dstp):
    return _sc_scatter_k()(msgp, dstp)


def _sc_scatter_body(msg_hbm, dst_hbm, out_hbm, idxd, vals, zb, acc):
    cid = lax.axis_index("c")
    sid = lax.axis_index("s")
    wid = sid * 2 + cid
    base = wid * EW
    _zero_rows(zb, SHARE4, 128)
    pltpu.sync_copy(zb, acc.at[pl.ds(sid * SHARE4, SHARE4)])
    plsc.subcore_barrier()
    for w in range(NWIN):
        lo = base + w * WIN
        pltpu.sync_copy(dst_hbm.at[pl.ds(lo, WIN)], idxd)
        pltpu.sync_copy(msg_hbm.at[pl.ds(lo, WIN)], vals)
        pltpu.sync_copy(vals, acc.at[idxd], add=True)
    plsc.subcore_barrier()
    pltpu.sync_copy(acc.at[pl.ds(sid * SHARE4, SHARE4)],
                    out_hbm.at[cid, pl.ds(sid * SHARE4, SHARE4)])


def _k_init_body(x_ref, wiT_ref, bi_ref, l0T_ref, l0b_ref, degp_ref,
                 h0_ref, hl0_ref, deg2_ref, cnti_ref):
    x = x_ref[...]
    h0 = jnp.dot(x, wiT_ref[...], preferred_element_type=f32) + bi_ref[...]
    h0_ref[...] = h0
    hl0_ref[...] = jnp.dot(h0, l0T_ref[...],
                           preferred_element_type=f32) + l0b_ref[...]
    degp = degp_ref[...]
    deg2 = degp[0:2, :] + degp[2:4, :]
    deg2_ref[...] = deg2
    cnti_ref[...] = 1.0 / jnp.maximum(deg2[0:1, :], 1.0)


def _tc_init(x, wiT, bi2, l0T, l0b2, degp4):
    return pl.pallas_call(
        _k_init_body,
        out_shape=(
            jax.ShapeDtypeStruct((N, H), f32),
            jax.ShapeDtypeStruct((N, H), f32),
            jax.ShapeDtypeStruct((2, NP), f32),
            jax.ShapeDtypeStruct((1, NP), f32),
        ),
    )(x, wiT, bi2, l0T, l0b2, degp4)


TE = 640
NT = E // TE


def _k_edge_body(xj_ref, i0_ref, i1_ref, ea_ref, dq_ref,
                 w10_ref, w11_ref, b1_ref, S_ref, M_ref, B2_ref,
                 e1T_ref, e1b_ref, e2T_ref, e2b_ref, out_ref):
    xj = xj_ref[...]
    i0 = i0_ref[...]
    i1 = i1_ref[...]
    t1 = jnp.maximum(i0 * w10_ref[...] + i1 * w11_ref[...] + b1_ref[...], 0.0)
    t1t = jnp.concatenate([t1] * H, axis=1)
    xjr = jnp.dot(xj, S_ref[...], preferred_element_type=f32)
    z = xjr * t1t
    tf = (jnp.dot(z, M_ref[...], preferred_element_type=f32)
          + jnp.dot(xj, B2_ref[...], preferred_element_type=f32))
    ef = jnp.maximum(
        jnp.dot(ea_ref[...], e1T_ref[...], preferred_element_type=f32)
        + e1b_ref[...], 0.0)
    ef = jnp.dot(ef, e2T_ref[...], preferred_element_type=f32) + e2b_ref[...]
    msg = tf + ef
    q = jnp.floor_divide(
        lax.broadcasted_iota(jnp.int32, (TE, 128), 1), 32).astype(f32)
    out_ref[...] = jnp.where(q == dq_ref[...],
                             jnp.concatenate([msg] * 4, axis=1), 0.0)


def _tc_edge(xj, inv0c, inv1c, ea, dq, w10, w11, b1, S, M, B2, e1T, e1b,
             e2T, e2b):
    full = lambda s: pl.BlockSpec(s, lambda i: (0, 0))
    return pl.pallas_call(
        _k_edge_body,
        grid=(NT,),
        in_specs=[
            pl.BlockSpec((TE, H), lambda i: (i, 0)),
            pl.BlockSpec((TE, 1), lambda i: (i, 0)),
            pl.BlockSpec((TE, 1), lambda i: (i, 0)),
            pl.BlockSpec((TE, DE), lambda i: (i, 0)),
            pl.BlockSpec((TE, 1), lambda i: (i, 0)),
            full((1, H)), full((1, H)), full((1, H)),
            full((H, H * H)), full((H * H, H)), full((H, H)),
            full((DE, H)), full((1, H)), full((H, H)), full((1, H)),
        ],
        out_specs=pl.BlockSpec((TE, 128), lambda i: (i, 0)),
        out_shape=jax.ShapeDtypeStruct((E, 128), f32),
    )(xj, inv0c, inv1c, ea, dq, w10, w11, b1, S, M, B2, e1T, e1b, e2T, e2b)


NBLK = 2000
NNB = N // NBLK


def _k_node_body(has_next, agg_a_ref, agg_b_ref, cnti_ref, hprev_ref,
                 batch_ref, a1T_ref, a1b_ref, a2r_ref, a2b_ref,
                 lnT_ref, lnb_ref,
                 h_ref, hl_ref, P_ref, ss_ref):
    agg = (agg_a_ref[...] + agg_b_ref[...]) * cnti_ref[...]
    h = jnp.maximum(agg, 0.0) + hprev_ref[...]
    h_ref[...] = h
    if has_next:
        hl_ref[...] = jnp.dot(h, lnT_ref[...],
                              preferred_element_type=f32) + lnb_ref[...]
    else:
        hl_ref[...] = jnp.zeros_like(h)
    t = jnp.tanh(jnp.dot(h, a1T_ref[...], preferred_element_type=f32)
                 + a1b_ref[...])
    s = jnp.sum(t * a2r_ref[...], axis=1, keepdims=True) + a2b_ref[...]
    es = jnp.exp(s)
    bf = batch_ref[...]
    p_rows, s_rows = [], []
    for b in range(B):
        oh = jnp.where(bf == float(b), es, 0.0)
        p_rows.append(jnp.sum(h * oh, axis=0, keepdims=True))
        s_rows.append(jnp.sum(oh, axis=0, keepdims=True))
    P_blk = jnp.concatenate(p_rows, axis=0)
    ss_blk = jnp.concatenate(s_rows, axis=0)

    @pl.when(pl.program_id(0) == 0)
    def _init():
        P_ref[...] = jnp.zeros_like(P_ref)
        ss_ref[...] = jnp.zeros_like(ss_ref)

    P_ref[...] += P_blk
    ss_ref[...] += ss_blk


def _tc_node(has_next, agg_a, agg_b, cnti, hprev, batchf,
             a1T, a1b, a2r, a2b, lnT, lnb):
    blk = lambda c: pl.BlockSpec((NBLK, c), lambda i: (i, 0))
    full = lambda r, c: pl.BlockSpec((r, c), lambda i: (0, 0))
    return pl.pallas_call(
        functools.partial(_k_node_body, has_next),
        grid=(NNB,),
        in_specs=[
            blk(H), blk(H), blk(1), blk(H), blk(1),
            full(H, 64), full(1, 64), full(1, 64), full(1, 1),
            full(H, H), full(1, H),
        ],
        out_specs=(
            pl.BlockSpec((NBLK, H), lambda i: (i, 0)),
            pl.BlockSpec((NBLK, H), lambda i: (i, 0)),
            full(B, H), full(B, 1),
        ),
        out_shape=(
            jax.ShapeDtypeStruct((N, H), f32),
            jax.ShapeDtypeStruct((N, H), f32),
            jax.ShapeDtypeStruct((B, H), f32),
            jax.ShapeDtypeStruct((B, 1), f32),
        ),
    )(agg_a, agg_b, cnti, hprev, batchf, a1T, a1b, a2r, a2b, lnT, lnb)


def _k_gru_body(P0_ref, ss0_ref, P1_ref, ss1_ref, trT_ref, trb_ref,
                wihT_ref, whhT_ref, bih_ref, bhh_ref,
                lnw_ref, lnb_ref, out_ref):
    g = (jnp.dot(P0_ref[...] / ss0_ref[...], trT_ref[...],
                 preferred_element_type=f32) + trb_ref[...])
    xg = (jnp.dot(P1_ref[...] / ss1_ref[...], trT_ref[...],
                  preferred_element_type=f32) + trb_ref[...])
    gi = jnp.dot(xg, wihT_ref[...], preferred_element_type=f32) + bih_ref[...]
    gh = jnp.dot(g, whhT_ref[...], preferred_element_type=f32) + bhh_ref[...]
    i_r, i_z, i_n = gi[:, 0:O], gi[:, O:2 * O], gi[:, 2 * O:3 * O]
    h_r, h_z, h_n = gh[:, 0:O], gh[:, O:2 * O], gh[:, 2 * O:3 * O]
    r = jax.nn.sigmoid(i_r + h_r)
    z = jax.nn.sigmoid(i_z + h_z)
    ng = jnp.tanh(i_n + r * h_n)
    g = (1.0 - z) * ng + z * g
    mu = jnp.mean(g, axis=1, keepdims=True)
    var = jnp.mean((g - mu) ** 2, axis=1, keepdims=True)
    out_ref[...] = ((g - mu) / jnp.sqrt(var + 1e-5) * lnw_ref[...]
                    + lnb_ref[...])


def _tc_gru(P0, ss0, P1, ss1, trT, trb, wihT, whhT, bih, bhh, lnw, lnb):
    return pl.pallas_call(
        _k_gru_body,
        out_shape=jax.ShapeDtypeStruct((B, O), f32),
    )(P0, ss0, P1, ss1, trT, trb, wihT, whhT, bih, bhh, lnw, lnb)


def kernel(x, edge_attr, edge_index, batch, Wi, bi, att1_W, att1_b, att2_W,
           att2_b, tr_W, tr_b, gru_Wih, gru_Whh, gru_bih, gru_bhh, ln_w,
           ln_b, lin_W0, lin_b0, ee1_W0, ee1_b0, ee2_W0, ee2_b0, tg1_W0,
           tg1_b0, tg2_W0, tg2_b0, lin_W1, lin_b1, ee1_W1, ee1_b1, ee2_W1,
           ee2_b1, tg1_W1, tg1_b1, tg2_W1, tg2_b1):
    src = edge_index[0]
    dst = edge_index[1]
    npad = EP - E
    trash = (N + (jnp.arange(npad, dtype=jnp.int32) % (NP - N))).astype(jnp.int32)
    srcp = jnp.concatenate([src, trash])
    dstp = jnp.concatenate([dst, trash])
    dst4 = dstp // 4
    dstq = (dst % 4).astype(f32).reshape(E, 1)

    degp = _sc_degrees(dstp, srcp)
    degp4 = degp.reshape(4, NP)

    h0, hl0, deg2, cnti_row = _tc_init(
        x, Wi.T, bi.reshape(1, H), lin_W0.T, lin_b0.reshape(1, H), degp4)
    degi = deg2[0]
    degj = deg2[1]
    cnti = cnti_row[0, :N].reshape(N, 1)

    hl0p = jnp.pad(hl0, ((0, NP - N), (0, 128 - H)))
    xj0, inv0, inv1 = _sc_gather0(hl0p, srcp, dstp, degi, degj)
    inv0c = inv0[:E].reshape(E, 1)
    inv1c = inv1[:E].reshape(E, 1)

    def edge_weights(tg1_W, tg1_b, tg2_W, tg2_b, ee1_W, ee1_b, ee2_W, ee2_b):
        w10 = tg1_W[:, 0].reshape(1, H)
        w11 = tg1_W[:, 1].reshape(1, H)
        b1 = tg1_b.reshape(1, H)
        S = jnp.repeat(jnp.eye(H, dtype=f32), H, axis=1)
        M = tg2_W.reshape(H, H, H).transpose(0, 2, 1).reshape(H * H, H)
        B2 = tg2_b.reshape(H, H)
        return (w10, w11, b1, S, M, B2, ee1_W.T, ee1_b.reshape(1, H),
                ee2_W.T, ee2_b.reshape(1, H))

    batchf = batch.astype(f32).reshape(N, 1)
    a1T = att1_W.T
    a1b = att1_b.reshape(1, 64)
    a2r = att2_W.reshape(1, 64)
    a2b = att2_b.reshape(1, 1)
    trT = tr_W.T
    trb = tr_b.reshape(1, O)

    ew0 = edge_weights(tg1_W0, tg1_b0, tg2_W0, tg2_b0,
                       ee1_W0, ee1_b0, ee2_W0, ee2_b0)
    msg0 = _tc_edge(xj0[:E, :H], inv0c, inv1c, edge_attr, dstq, *ew0)
    msg0p = jnp.pad(msg0, ((0, EP - E), (0, 0)))
    aggp0 = _sc_scatter(msg0p, dst4).reshape(2, NP, H)
    h1, hl1, P0, ss0 = _tc_node(True, aggp0[0, :N], aggp0[1, :N], cnti, h0,
                                batchf, a1T, a1b, a2r, a2b,
                                lin_W1.T, lin_b1.reshape(1, H))

    hl1p = jnp.pad(hl1, ((0, NP - N), (0, 128 - H)))
    xj1 = _sc_gather1(hl1p, srcp)
    ew1 = edge_weights(tg1_W1, tg1_b1, tg2_W1, tg2_b1,
                       ee1_W1, ee1_b1, ee2_W1, ee2_b1)
    msg1 = _tc_edge(xj1[:E, :H], inv0c, inv1c, edge_attr, dstq, *ew1)
    msg1p = jnp.pad(msg1, ((0, EP - E), (0, 0)))
    aggp1 = _sc_scatter(msg1p, dst4).reshape(2, NP, H)
    h2, _, P1, ss1 = _tc_node(False, aggp1[0, :N], aggp1[1, :N], cnti, h1,
                              batchf, a1T, a1b, a2r, a2b,
                              lin_W1.T, lin_b1.reshape(1, H))

    return _tc_gru(P0, ss0, P1, ss1, trT, trb, gru_Wih.T, gru_Whh.T,
                   gru_bih.reshape(1, 3 * O), gru_bhh.reshape(1, 3 * O),
                   ln_w.reshape(1, O), ln_b.reshape(1, O))
